# Initial kernel scaffold; baseline (speedup 1.0000x reference)
import jax
import jax.numpy as jnp
from jax.experimental import pallas as pl


def kernel(logits, edge_index):
    raise NotImplementedError("write your pallas kernel here")


# speedup vs baseline: 4.9449x; 4.9449x over previous
import functools

import jax
import jax.numpy as jnp
from jax import lax
from jax.experimental import pallas as pl
from jax.experimental.pallas import tpu as pltpu
from jax.experimental.pallas import tpu_sc as plsc

N_NODES = 100000
N_HEADS = 8
N_EDGES = 6400000
NSHARD = 4


@functools.lru_cache(maxsize=None)
def _build(E, H, N, nshard, C, CM, CP, interpret=False):
    ES = E // nshard
    NCHUNK = ES // C
    NV = C // 16
    NTV = N // 16
    mesh = plsc.VectorSubcoreMesh(
        core_axis_name="c", subcore_axis_name="s", num_cores=2, num_subcores=16
    )
    NW = 2 * 16
    P = (nshard * H * N) // NW

    def tile_ids():
        c = lax.axis_index("c")
        s = lax.axis_index("s")
        head = s % H
        shard = (s // H) * 2 + c
        return c, s, head, shard

    @functools.partial(
        pl.kernel,
        out_type=jax.ShapeDtypeStruct((nshard, H, N), jnp.float32),
        mesh=mesh,
        scratch_types=[
            pltpu.VMEM((N,), jnp.float32),
            pltpu.VMEM((C,), jnp.int32),
            pltpu.VMEM((C,), jnp.float32),
        ],
        compiler_params=pltpu.CompilerParams(needs_layout_passes=False, use_tc_tiling_on_sc=False),
        interpret=interpret,
    )
    def k1(lt_hbm, dst_hbm, pmax_hbm, tab, dbuf, lbuf):
        _, _, head, shard = tile_ids()
        base = shard * ES

        neg_inf = jnp.full((16,), -jnp.inf, jnp.float32)

        def init_body(i, _):
            tab[pl.ds(i * 16, 16)] = neg_inf
            return 0

        lax.fori_loop(0, NTV, init_body, 0)

        def chunk_body(g, _):
            off = base + g * C
            pltpu.sync_copy(dst_hbm.at[pl.ds(off, C)], dbuf)
            pltpu.sync_copy(lt_hbm.at[head, pl.ds(off, C)], lbuf)

            def vreg_body(v, _):
                kd = dbuf[pl.ds(v * 16, 16)]
                x = lbuf[pl.ds(v * 16, 16)]
                sv, sk = plsc.sort_key_val(x, kd)
                _, mlast = plsc.scan_count(sk)
                old = plsc.load_gather(tab, [sk], mask=mlast)
                plsc.store_scatter(tab, [sk], jnp.maximum(old, sv), mask=mlast)
                return 0

            lax.fori_loop(0, NV, vreg_body, 0)
            return 0

        lax.fori_loop(0, NCHUNK, chunk_body, 0)
        pltpu.sync_copy(tab, pmax_hbm.at[shard, head])

    NM = N // CM
    NMV = CM // 16

    @functools.partial(
        pl.kernel,
        out_type=jax.ShapeDtypeStruct((H, E), jnp.float32),
        mesh=mesh,
        scratch_types=[
            pltpu.VMEM((N,), jnp.float32),
            pltpu.VMEM((nshard, CM), jnp.float32),
            pltpu.VMEM((C,), jnp.int32),
            pltpu.VMEM((C,), jnp.float32),
            pltpu.VMEM((C,), jnp.float32),
        ],
        compiler_params=pltpu.CompilerParams(needs_layout_passes=False, use_tc_tiling_on_sc=False),
        interpret=interpret,
    )
    def k3a(lt_hbm, dst_hbm, pmax_hbm, sco_hbm, tab, mbuf, dbuf, lbuf, sbuf):
        _, _, head, shard = tile_ids()
        base = shard * ES

        def merge_body(t, _):
            noff = t * CM
            for j in range(nshard):
                pltpu.sync_copy(pmax_hbm.at[j, head, pl.ds(noff, CM)], mbuf.at[j])

            def mv_body(v, _):
                m = mbuf[0, pl.ds(v * 16, 16)]
                for j in range(1, nshard):
                    m = jnp.maximum(m, mbuf[j, pl.ds(v * 16, 16)])
                tab[pl.ds(noff + v * 16, 16)] = m
                return 0

            lax.fori_loop(0, NMV, mv_body, 0)
            return 0

        lax.fori_loop(0, NM, merge_body, 0)

        def chunk_body(g, _):
            off = base + g * C
            pltpu.sync_copy(dst_hbm.at[pl.ds(off, C)], dbuf)
            pltpu.sync_copy(lt_hbm.at[head, pl.ds(off, C)], lbuf)

            def vreg_body(v, _):
                kd = dbuf[pl.ds(v * 16, 16)]
                x = lbuf[pl.ds(v * 16, 16)]
                mx = plsc.load_gather(tab, [kd])
                sbuf[pl.ds(v * 16, 16)] = jnp.exp(x - mx)
                return 0

            lax.fori_loop(0, NV, vreg_body, 0)
            pltpu.sync_copy(sbuf, sco_hbm.at[head, pl.ds(off, C)])
            return 0

        lax.fori_loop(0, NCHUNK, chunk_body, 0)

    @functools.partial(
        pl.kernel,
        out_type=jax.ShapeDtypeStruct((nshard, H, N), jnp.float32),
        mesh=mesh,
        scratch_types=[
            pltpu.VMEM((N,), jnp.float32),
            pltpu.VMEM((C,), jnp.int32),
            pltpu.VMEM((C,), jnp.float32),
        ],
        compiler_params=pltpu.CompilerParams(needs_layout_passes=False, use_tc_tiling_on_sc=False),
        interpret=interpret,
    )
    def k3b(sco_hbm, dst_hbm, pnorm_hbm, tab, dbuf, sbuf):
        _, _, head, shard = tile_ids()
        base = shard * ES

        zeros = jnp.zeros((16,), jnp.float32)

        def init_body(i, _):
            tab[pl.ds(i * 16, 16)] = zeros
            return 0

        lax.fori_loop(0, NTV, init_body, 0)

        def chunk_body(g, _):
            off = base + g * C
            pltpu.sync_copy(dst_hbm.at[pl.ds(off, C)], dbuf)
            pltpu.sync_copy(sco_hbm.at[head, pl.ds(off, C)], sbuf)

            def vreg_body(v, _):
                kd = dbuf[pl.ds(v * 16, 16)]
                sval = sbuf[pl.ds(v * 16, 16)]
                plsc.addupdate_scatter(tab, [kd], sval)
                return 0

            lax.fori_loop(0, NV, vreg_body, 0)
            return 0

        lax.fori_loop(0, NCHUNK, chunk_body, 0)
        pltpu.sync_copy(tab, pnorm_hbm.at[shard, head])

    FL = H * N
    SPAN = FL // NW
    NP = SPAN // CP
    NFV = CP // 16
    TAIL = CP - NFV * 16

    @functools.partial(
        pl.kernel,
        out_type=jax.ShapeDtypeStruct((FL,), jnp.float32),
        mesh=mesh,
        scratch_types=[
            pltpu.VMEM((nshard, CP), jnp.float32),
            pltpu.VMEM((CP,), jnp.float32),
        ],
        compiler_params=pltpu.CompilerParams(needs_layout_passes=False, use_tc_tiling_on_sc=False),
        interpret=interpret,
    )
    def k4(pnorm_hbm, norm_hbm, abuf, obuf):
        c = lax.axis_index("c")
        s = lax.axis_index("s")
        tid = s * 2 + c
        toff = tid * SPAN

        def chunk_body(t, _):
            off = toff + t * CP
            for j in range(nshard):
                pltpu.sync_copy(pnorm_hbm.at[j, pl.ds(off, CP)], abuf.at[j])

            def v_body(v, _):
                acc = abuf[0, pl.ds(v * 16, 16)]
                for j in range(1, nshard):
                    acc = acc + abuf[j, pl.ds(v * 16, 16)]
                obuf[pl.ds(v * 16, 16)] = acc
                return 0

            lax.fori_loop(0, NFV, v_body, 0)
            if TAIL:
                tv = CP - 16
                acc = abuf[0, pl.ds(tv, 16)]
                for j in range(1, nshard):
                    acc = acc + abuf[j, pl.ds(tv, 16)]
                obuf[pl.ds(tv, 16)] = acc
            pltpu.sync_copy(obuf, norm_hbm.at[pl.ds(off, CP)])
            return 0

        lax.fori_loop(0, NP, chunk_body, 0)

    return k1, k3a, k3b, k4


def kernel(logits, edge_index):
    E, H, _ = logits.shape
    N = N_NODES
    dst = edge_index[1].astype(jnp.int32)
    lt = jnp.transpose(logits.reshape(E, H))
    k1, k3a, k3b, k4 = _build(E, H, N, NSHARD, 6400, 2000, 5000)
    pmax = k1(lt, dst)
    scoresT = k3a(lt, dst, pmax)
    pnorm = k3b(scoresT, dst)
    norm = k4(pnorm.reshape(NSHARD, H * N))
    scores = jnp.transpose(scoresT).reshape(E, H, 1)
    normalizer = jnp.transpose(norm.reshape(H, N)).reshape(N, H, 1)
    return scores, normalizer
